# baseline (speedup 1.0000x reference)
import functools

import jax
import jax.numpy as jnp
from jax import lax
from jax.experimental import pallas as pl
from jax.experimental.pallas import tpu as pltpu
from jax.experimental.pallas import tpu_sc as plsc

_A = 20000
_B = 8
_TOP_N = 200
_IOU_THR = 0.7
_LANES = 128

_A_PAD = 20480
_ROWS = _A_PAD // _LANES
_KCAP = 208
_L1N = _A_PAD // 16
_L2N = _L1N // 16
_BIG = 1 << 30
_NEG_INF = float("-inf")


def _roll_m2(x):
    return jnp.concatenate([x[:, 2:], x[:, :2]], axis=1)


def _roll_p2(x):
    return jnp.concatenate([x[:, -2:], x[:, :-2]], axis=1)


def _decode_body(b, cls_ref, reg_ref, anch_ref, sc_ref, box_ref):
    sc_ref[...] = jax.nn.sigmoid(cls_ref[...])

    rows = anch_ref.shape[0]
    lane = lax.broadcasted_iota(jnp.int32, (rows, _LANES), 1)
    is01 = (lane % 4) < 2
    a = anch_ref[...]
    h = _roll_m2(a) - a
    ctr = a + 0.5 * h
    h2 = _roll_p2(h)
    for i in range(b):
        d = reg_ref[i]
        pc = d * h + ctr
        sz = jnp.exp(d) * h2
        lo = pc - 0.5 * _roll_m2(sz)
        hi = _roll_p2(pc) + 0.5 * sz
        box_ref[i] = jnp.clip(jnp.where(is01, lo, hi), 0.0, 1.0)


def _decode_tc(rpn_cls, rpn_reg, anchors, b, a_valid, interpret=False):
    crows = b * a_valid // _LANES
    rrows = 4 * a_valid // _LANES
    cls_f = rpn_cls.reshape(crows, _LANES)
    reg_f = rpn_reg.reshape(b, rrows, _LANES)
    anch_f = anchors.reshape(rrows, _LANES)

    scores, boxes = pl.pallas_call(
        functools.partial(_decode_body, b),
        out_shape=[
            jax.ShapeDtypeStruct((crows, _LANES), jnp.float32),
            jax.ShapeDtypeStruct((b, rrows, _LANES), jnp.float32),
        ],
        interpret=interpret,
    )(cls_f, reg_f, anch_f)
    return scores.reshape(b * a_valid), boxes.reshape(b, 4 * a_valid)


def _axis_ids():
    return lax.axis_index("c"), lax.axis_index("s")


def _store1(ref, pos, val):
    plsc.store_scatter(ref, [jnp.full((16,), pos, jnp.int32)],
                       jnp.full((16,), val, jnp.float32),
                       mask=lax.iota(jnp.int32, 16) == 0)


def _sc_nms_body(top_n, iou_thr, n_samples, a_valid,
                 sc_hbm, box_hbm, out_hbm,
                 sc_v, box_v, l1_v, l2_v,
                 ky0_v, kx0_v, ky1_v, kx1_v, kar_v, sem_a, sem_b):
    c, s = _axis_ids()
    n_cores = 2
    per_core = n_samples // n_cores

    @pl.when(s < per_core)
    def _work():
        samp = c * per_core + s
        h_sc = pltpu.async_copy(sc_hbm.at[pl.ds(samp * a_valid, a_valid)],
                                sc_v.at[pl.ds(0, a_valid)], sem_a)
        h_bx = pltpu.async_copy(box_hbm.at[samp], box_v, sem_b)

        iota = lax.iota(jnp.int32, 16)
        zeros16 = jnp.zeros((16,), jnp.float32)
        neg16 = jnp.full((16,), _NEG_INF, jnp.float32)

        for j in range((_A_PAD - a_valid) // 16):
            sc_v[pl.ds(a_valid + j * 16, 16)] = neg16

        for j in range(_KCAP // 16):
            ky0_v[pl.ds(j * 16, 16)] = zeros16
            kx0_v[pl.ds(j * 16, 16)] = zeros16
            ky1_v[pl.ds(j * 16, 16)] = zeros16
            kx1_v[pl.ds(j * 16, 16)] = zeros16
            kar_v[pl.ds(j * 16, 16)] = zeros16

        h_sc.wait()

        def l1_build(i, _):
            for u in range(4):
                ch = sc_v[pl.ds((i * 4 + u) * 16, 16)]
                _store1(l1_v, i * 4 + u, jnp.max(ch))
            return 0
        lax.fori_loop(0, _L1N // 4, l1_build, 0)

        def l2_build(i, _):
            for u in range(4):
                ch = l1_v[pl.ds((i * 4 + u) * 16, 16)]
                _store1(l2_v, i * 4 + u, jnp.max(ch))
            return 0
        lax.fori_loop(0, _L2N // 4, l2_build, 0)

        h_bx.wait()

        def wcond(state):
            kn, alive = state
            return (kn < top_n) & (alive > 0)

        def wbody(state):
            kn, alive = state

            l2chunks = [l2_v[pl.ds(k * 16, 16)] for k in range(_L2N // 16)]
            vmax = l2chunks[0]
            for ch in l2chunks[1:]:
                vmax = jnp.maximum(vmax, ch)
            m = jnp.max(vmax)

            cand = jnp.where(l2chunks[0] == m, iota, _BIG)
            for k, ch in enumerate(l2chunks[1:]):
                cand = jnp.minimum(
                    cand, jnp.where(ch == m, (k + 1) * 16 + iota, _BIG))
            p2 = jnp.min(cand)

            ch1 = l1_v[pl.ds(p2 * 16, 16)]
            f1 = plsc.all_reduce_ffs(ch1 == m)
            p1 = p2 * 16 + f1[0]

            ch0 = sc_v[pl.ds(p1 * 16, 16)]
            f0 = plsc.all_reduce_ffs(ch0 == m)
            hit0 = iota == f0

            live = m > _NEG_INF

            @pl.when(live)
            def _consume():
                ch0n = jnp.where(hit0, _NEG_INF, ch0)
                sc_v[pl.ds(p1 * 16, 16)] = ch0n
                m1 = jnp.max(ch0n)
                _store1(l1_v, p1, m1)
                ch1n = jnp.where(iota == f1, m1, ch1)
                _store1(l2_v, p2, jnp.max(ch1n))

            pvec = jnp.minimum(p1 * 16 + f0, a_valid - 1) * 4
            cy0 = plsc.load_gather(box_v, [pvec])
            cx0 = plsc.load_gather(box_v, [pvec + 1])
            cy1 = plsc.load_gather(box_v, [pvec + 2])
            cx1 = plsc.load_gather(box_v, [pvec + 3])
            car = (cy1 - cy0) * (cx1 - cx0)

            nkc = (kn + 15) // 16

            def ibody(j, mxv):
                a0 = ky0_v[pl.ds(j * 16, 16)]
                b0 = kx0_v[pl.ds(j * 16, 16)]
                a1 = ky1_v[pl.ds(j * 16, 16)]
                b1 = kx1_v[pl.ds(j * 16, 16)]
                ar = kar_v[pl.ds(j * 16, 16)]
                yi0 = jnp.maximum(cy0, a0)
                xi0 = jnp.maximum(cx0, b0)
                yi1 = jnp.minimum(cy1, a1)
                xi1 = jnp.minimum(cx1, b1)
                inter = (jnp.maximum(yi1 - yi0, 0.0)
                         * jnp.maximum(xi1 - xi0, 0.0))
                iou = inter / (car + ar - inter + 1e-8)
                return jnp.maximum(mxv, iou)

            mxv = lax.fori_loop(0, nkc, ibody,
                                jnp.full((16,), _NEG_INF, jnp.float32))
            keep = live & (jnp.max(mxv) <= iou_thr)

            @pl.when(keep)
            def _append():
                _store1(ky0_v, kn, cy0[0])
                _store1(kx0_v, kn, cx0[0])
                _store1(ky1_v, kn, cy1[0])
                _store1(kx1_v, kn, cx1[0])
                _store1(kar_v, kn, car[0])

            kn = kn + jnp.where(keep, jnp.int32(1), jnp.int32(0))
            return (kn, jnp.where(live, jnp.int32(1), jnp.int32(0)))

        kn, _ = lax.while_loop(wcond, wbody,
                               (jnp.int32(0), jnp.int32(1)))

        v0 = box_v[pl.ds(0, 16)]
        b0y0 = jnp.full((16,), v0[0], jnp.float32)
        b0x0 = jnp.full((16,), v0[1], jnp.float32)
        b0y1 = jnp.full((16,), v0[2], jnp.float32)
        b0x1 = jnp.full((16,), v0[3], jnp.float32)
        for j in range(_KCAP // 16):
            kidx = j * 16 + iota
            mask = kidx >= kn
            ky0_v[pl.ds(j * 16, 16)] = jnp.where(
                mask, b0y0, ky0_v[pl.ds(j * 16, 16)])
            kx0_v[pl.ds(j * 16, 16)] = jnp.where(
                mask, b0x0, kx0_v[pl.ds(j * 16, 16)])
            ky1_v[pl.ds(j * 16, 16)] = jnp.where(
                mask, b0y1, ky1_v[pl.ds(j * 16, 16)])
            kx1_v[pl.ds(j * 16, 16)] = jnp.where(
                mask, b0x1, kx1_v[pl.ds(j * 16, 16)])

        pltpu.sync_copy(ky0_v, out_hbm.at[samp, 0])
        pltpu.sync_copy(kx0_v, out_hbm.at[samp, 1])
        pltpu.sync_copy(ky1_v, out_hbm.at[samp, 2])
        pltpu.sync_copy(kx1_v, out_hbm.at[samp, 3])


def _sc_nms(scores, boxes, b, a_valid, top_n, iou_thr, interpret=False):
    mesh = plsc.VectorSubcoreMesh(core_axis_name="c", subcore_axis_name="s",
                                  num_cores=2, num_subcores=16)
    fn = pl.kernel(
        functools.partial(_sc_nms_body, top_n, iou_thr, b, a_valid),
        out_type=jax.ShapeDtypeStruct((b, 4, _KCAP), jnp.float32),
        mesh=mesh,
        scratch_types=[
            pltpu.VMEM((_A_PAD,), jnp.float32),
            pltpu.VMEM((4 * a_valid,), jnp.float32),
            pltpu.VMEM((_L1N,), jnp.float32),
            pltpu.VMEM((_L2N,), jnp.float32),
            pltpu.VMEM((_KCAP,), jnp.float32),
            pltpu.VMEM((_KCAP,), jnp.float32),
            pltpu.VMEM((_KCAP,), jnp.float32),
            pltpu.VMEM((_KCAP,), jnp.float32),
            pltpu.VMEM((_KCAP,), jnp.float32),
            pltpu.SemaphoreType.DMA,
            pltpu.SemaphoreType.DMA,
        ],
        compiler_params=pltpu.CompilerParams(needs_layout_passes=False),
        interpret=interpret,
    )
    return fn(scores, boxes)


def _proposal(rpn_cls, rpn_reg, anchors, top_n, iou_thr, interpret=False):
    b = rpn_cls.shape[0]
    a = rpn_cls.shape[1]
    scores, boxes = _decode_tc(rpn_cls, rpn_reg, anchors, b, a,
                               interpret=interpret)
    out = _sc_nms(scores, boxes, b, a, top_n, iou_thr,
                  interpret=interpret)
    proposals = jnp.transpose(out, (0, 2, 1))[:, :top_n, :].reshape(
        b * top_n, 4)
    indices = jnp.zeros((b * top_n,), jnp.int32)
    return proposals, indices


def kernel(rpn_cls, rpn_reg, anchors):
    return _proposal(rpn_cls, rpn_reg, anchors, _TOP_N, _IOU_THR)


# speedup vs baseline: 1.6975x; 1.6975x over previous
import functools

import jax
import jax.numpy as jnp
from jax import lax
from jax.experimental import pallas as pl
from jax.experimental.pallas import tpu as pltpu
from jax.experimental.pallas import tpu_sc as plsc

_A = 20000
_B = 8
_TOP_N = 200
_IOU_THR = 0.7
_LANES = 128

_A_PAD = 20480
_ROWS = _A_PAD // _LANES
_KCAP = 208
_L1N = _A_PAD // 16
_L2N = _L1N // 16
_BIG = 1 << 30
_NEG_INF = float("-inf")


def _decode_body(a_valid, cls_ref, reg_ref, anch_ref,
                 sc_ref, y0_ref, x0_ref, y1_ref, x1_ref):
    row_iota = lax.broadcasted_iota(jnp.int32, (_ROWS, _LANES), 0)
    col_iota = lax.broadcasted_iota(jnp.int32, (_ROWS, _LANES), 1)
    valid = (row_iota * _LANES + col_iota) < a_valid

    scores = jax.nn.sigmoid(cls_ref[0])
    sc_ref[0] = jnp.where(valid, scores, _NEG_INF)

    aymin = anch_ref[0]
    axmin = anch_ref[1]
    aymax = anch_ref[2]
    axmax = anch_ref[3]
    ah = aymax - aymin
    aw = axmax - axmin
    acy = aymin + 0.5 * ah
    acx = axmin + 0.5 * aw
    dy = reg_ref[0, 0]
    dx = reg_ref[0, 1]
    dh = reg_ref[0, 2]
    dw = reg_ref[0, 3]
    pcy = dy * ah + acy
    pcx = dx * aw + acx
    ph = jnp.exp(dh) * ah
    pw = jnp.exp(dw) * aw
    y0_ref[0] = jnp.clip(pcy - 0.5 * ph, 0.0, 1.0)
    x0_ref[0] = jnp.clip(pcx - 0.5 * pw, 0.0, 1.0)
    y1_ref[0] = jnp.clip(pcy + 0.5 * ph, 0.0, 1.0)
    x1_ref[0] = jnp.clip(pcx + 0.5 * pw, 0.0, 1.0)


def _decode_tc(rpn_cls, rpn_reg, anchors, b, a_valid, interpret=False):
    pad = _A_PAD - a_valid
    cls_p = jnp.pad(rpn_cls[..., 0], ((0, 0), (0, pad))).reshape(
        b, _ROWS, _LANES)
    reg_p = jnp.pad(jnp.transpose(rpn_reg, (0, 2, 1)),
                    ((0, 0), (0, 0), (0, pad))).reshape(b, 4, _ROWS, _LANES)
    anch_p = jnp.pad(jnp.transpose(anchors, (1, 0)),
                     ((0, 0), (0, pad))).reshape(4, _ROWS, _LANES)

    plane = jax.ShapeDtypeStruct((b, _ROWS, _LANES), jnp.float32)
    outs = pl.pallas_call(
        functools.partial(_decode_body, a_valid),
        grid=(b,),
        in_specs=[
            pl.BlockSpec((1, _ROWS, _LANES), lambda i: (i, 0, 0)),
            pl.BlockSpec((1, 4, _ROWS, _LANES), lambda i: (i, 0, 0, 0)),
            pl.BlockSpec((4, _ROWS, _LANES), lambda i: (0, 0, 0)),
        ],
        out_specs=[pl.BlockSpec((1, _ROWS, _LANES), lambda i: (i, 0, 0))] * 5,
        out_shape=[plane] * 5,
        interpret=interpret,
    )(cls_p, reg_p, anch_p)
    return [o.reshape(b, _A_PAD) for o in outs]


def _axis_ids():
    return lax.axis_index("c"), lax.axis_index("s")


def _store1(ref, pos, val):
    plsc.store_scatter(ref, [jnp.full((16,), pos, jnp.int32)],
                       jnp.full((16,), val, jnp.float32),
                       mask=lax.iota(jnp.int32, 16) == 0)


def _sc_nms_body(top_n, iou_thr, n_samples,
                 sc_hbm, y0_hbm, x0_hbm, y1_hbm, x1_hbm, out_hbm,
                 sc_v, y0_v, x0_v, y1_v, x1_v, l1_v, l2_v,
                 ky0_v, kx0_v, ky1_v, kx1_v, kar_v, sem_a, sem_b):
    c, s = _axis_ids()
    n_cores = 2
    per_core = n_samples // n_cores

    @pl.when(s < per_core)
    def _work():
        samp = c * per_core + s
        h_sc = pltpu.async_copy(sc_hbm.at[samp], sc_v, sem_a)
        h_y0 = pltpu.async_copy(y0_hbm.at[samp], y0_v, sem_b)
        h_x0 = pltpu.async_copy(x0_hbm.at[samp], x0_v, sem_b)
        h_y1 = pltpu.async_copy(y1_hbm.at[samp], y1_v, sem_b)
        h_x1 = pltpu.async_copy(x1_hbm.at[samp], x1_v, sem_b)

        iota = lax.iota(jnp.int32, 16)
        zeros16 = jnp.zeros((16,), jnp.float32)

        for j in range(_KCAP // 16):
            ky0_v[pl.ds(j * 16, 16)] = zeros16
            kx0_v[pl.ds(j * 16, 16)] = zeros16
            ky1_v[pl.ds(j * 16, 16)] = zeros16
            kx1_v[pl.ds(j * 16, 16)] = zeros16
            kar_v[pl.ds(j * 16, 16)] = zeros16

        h_sc.wait()

        def l1_build(i, _):
            for u in range(4):
                ch = sc_v[pl.ds((i * 4 + u) * 16, 16)]
                _store1(l1_v, i * 4 + u, jnp.max(ch))
            return 0
        lax.fori_loop(0, _L1N // 4, l1_build, 0)

        def l2_build(i, _):
            for u in range(4):
                ch = l1_v[pl.ds((i * 4 + u) * 16, 16)]
                _store1(l2_v, i * 4 + u, jnp.max(ch))
            return 0
        lax.fori_loop(0, _L2N // 4, l2_build, 0)

        h_y0.wait()
        h_x0.wait()
        h_y1.wait()
        h_x1.wait()

        def wcond(state):
            kn, alive = state
            return (kn < top_n) & (alive > 0)

        def wbody(state):
            kn, alive = state

            l2chunks = [l2_v[pl.ds(k * 16, 16)] for k in range(_L2N // 16)]
            vmax = l2chunks[0]
            for ch in l2chunks[1:]:
                vmax = jnp.maximum(vmax, ch)
            m = jnp.max(vmax)

            cand = jnp.where(l2chunks[0] == m, iota, _BIG)
            for k, ch in enumerate(l2chunks[1:]):
                cand = jnp.minimum(
                    cand, jnp.where(ch == m, (k + 1) * 16 + iota, _BIG))
            p2 = jnp.min(cand)

            ch1 = l1_v[pl.ds(p2 * 16, 16)]
            f1 = plsc.all_reduce_ffs(ch1 == m)
            p1 = p2 * 16 + f1[0]

            ch0 = sc_v[pl.ds(p1 * 16, 16)]
            f0 = plsc.all_reduce_ffs(ch0 == m)
            hit0 = iota == f0

            live = m > _NEG_INF

            @pl.when(live)
            def _consume():
                ch0n = jnp.where(hit0, _NEG_INF, ch0)
                sc_v[pl.ds(p1 * 16, 16)] = ch0n
                m1 = jnp.max(ch0n)
                _store1(l1_v, p1, m1)
                ch1n = jnp.where(iota == f1, m1, ch1)
                _store1(l2_v, p2, jnp.max(ch1n))

            pvec = p1 * 16 + f0
            cy0 = plsc.load_gather(y0_v, [pvec])
            cx0 = plsc.load_gather(x0_v, [pvec])
            cy1 = plsc.load_gather(y1_v, [pvec])
            cx1 = plsc.load_gather(x1_v, [pvec])
            car = (cy1 - cy0) * (cx1 - cx0)

            nkc = (kn + 15) // 16

            def ibody(j, mxv):
                a0 = ky0_v[pl.ds(j * 16, 16)]
                b0 = kx0_v[pl.ds(j * 16, 16)]
                a1 = ky1_v[pl.ds(j * 16, 16)]
                b1 = kx1_v[pl.ds(j * 16, 16)]
                ar = kar_v[pl.ds(j * 16, 16)]
                yi0 = jnp.maximum(cy0, a0)
                xi0 = jnp.maximum(cx0, b0)
                yi1 = jnp.minimum(cy1, a1)
                xi1 = jnp.minimum(cx1, b1)
                inter = (jnp.maximum(yi1 - yi0, 0.0)
                         * jnp.maximum(xi1 - xi0, 0.0))
                iou = inter / (car + ar - inter + 1e-8)
                return jnp.maximum(mxv, iou)

            mxv = lax.fori_loop(0, nkc, ibody,
                                jnp.full((16,), _NEG_INF, jnp.float32))
            keep = live & (jnp.max(mxv) <= iou_thr)

            @pl.when(keep)
            def _append():
                _store1(ky0_v, kn, cy0[0])
                _store1(kx0_v, kn, cx0[0])
                _store1(ky1_v, kn, cy1[0])
                _store1(kx1_v, kn, cx1[0])
                _store1(kar_v, kn, car[0])

            kn = kn + jnp.where(keep, jnp.int32(1), jnp.int32(0))
            return (kn, jnp.where(live, jnp.int32(1), jnp.int32(0)))

        kn, _ = lax.while_loop(wcond, wbody,
                               (jnp.int32(0), jnp.int32(1)))

        b0y0 = jnp.full((16,), y0_v[pl.ds(0, 16)][0], jnp.float32)
        b0x0 = jnp.full((16,), x0_v[pl.ds(0, 16)][0], jnp.float32)
        b0y1 = jnp.full((16,), y1_v[pl.ds(0, 16)][0], jnp.float32)
        b0x1 = jnp.full((16,), x1_v[pl.ds(0, 16)][0], jnp.float32)
        for j in range(_KCAP // 16):
            kidx = j * 16 + iota
            mask = kidx >= kn
            ky0_v[pl.ds(j * 16, 16)] = jnp.where(
                mask, b0y0, ky0_v[pl.ds(j * 16, 16)])
            kx0_v[pl.ds(j * 16, 16)] = jnp.where(
                mask, b0x0, kx0_v[pl.ds(j * 16, 16)])
            ky1_v[pl.ds(j * 16, 16)] = jnp.where(
                mask, b0y1, ky1_v[pl.ds(j * 16, 16)])
            kx1_v[pl.ds(j * 16, 16)] = jnp.where(
                mask, b0x1, kx1_v[pl.ds(j * 16, 16)])

        pltpu.sync_copy(ky0_v, out_hbm.at[samp, 0])
        pltpu.sync_copy(kx0_v, out_hbm.at[samp, 1])
        pltpu.sync_copy(ky1_v, out_hbm.at[samp, 2])
        pltpu.sync_copy(kx1_v, out_hbm.at[samp, 3])


def _sc_nms(scores, y0, x0, y1, x1, b, top_n, iou_thr, interpret=False):
    mesh = plsc.VectorSubcoreMesh(core_axis_name="c", subcore_axis_name="s",
                                  num_cores=2, num_subcores=16)
    fn = pl.kernel(
        functools.partial(_sc_nms_body, top_n, iou_thr, b),
        out_type=jax.ShapeDtypeStruct((b, 4, _KCAP), jnp.float32),
        mesh=mesh,
        scratch_types=[
            pltpu.VMEM((_A_PAD,), jnp.float32),
            pltpu.VMEM((_A_PAD,), jnp.float32),
            pltpu.VMEM((_A_PAD,), jnp.float32),
            pltpu.VMEM((_A_PAD,), jnp.float32),
            pltpu.VMEM((_A_PAD,), jnp.float32),
            pltpu.VMEM((_L1N,), jnp.float32),
            pltpu.VMEM((_L2N,), jnp.float32),
            pltpu.VMEM((_KCAP,), jnp.float32),
            pltpu.VMEM((_KCAP,), jnp.float32),
            pltpu.VMEM((_KCAP,), jnp.float32),
            pltpu.VMEM((_KCAP,), jnp.float32),
            pltpu.VMEM((_KCAP,), jnp.float32),
            pltpu.SemaphoreType.DMA,
            pltpu.SemaphoreType.DMA,
        ],
        compiler_params=pltpu.CompilerParams(needs_layout_passes=False,
                                             skip_device_barrier=True),
        interpret=interpret,
    )
    return fn(scores, y0, x0, y1, x1)


def _proposal(rpn_cls, rpn_reg, anchors, top_n, iou_thr, interpret=False):
    b = rpn_cls.shape[0]
    a = rpn_cls.shape[1]
    scores, y0, x0, y1, x1 = _decode_tc(rpn_cls, rpn_reg, anchors, b, a,
                                        interpret=interpret)
    out = _sc_nms(scores, y0, x0, y1, x1, b, top_n, iou_thr,
                  interpret=interpret)
    proposals = jnp.transpose(out, (0, 2, 1))[:, :top_n, :].reshape(
        b * top_n, 4)
    indices = jnp.zeros((b * top_n,), jnp.int32)
    return proposals, indices


def kernel(rpn_cls, rpn_reg, anchors):
    return _proposal(rpn_cls, rpn_reg, anchors, _TOP_N, _IOU_THR)


# speedup vs baseline: 1.7120x; 1.0086x over previous
import functools

import jax
import jax.numpy as jnp
from jax import lax
from jax.experimental import pallas as pl
from jax.experimental.pallas import tpu as pltpu
from jax.experimental.pallas import tpu_sc as plsc

_A = 20000
_B = 8
_TOP_N = 200
_IOU_THR = 0.7
_LANES = 128

_A_PAD = 20480
_ROWS = _A_PAD // _LANES
_KCAP = 208
_L1N = _A_PAD // 16
_L2N = _L1N // 16
_BIG = 1 << 30
_NEG_INF = float("-inf")


def _decode_body(a_valid, cls_ref, reg_ref, anch_ref,
                 sc_ref, y0_ref, x0_ref, y1_ref, x1_ref):
    row_iota = lax.broadcasted_iota(jnp.int32, (_ROWS, _LANES), 0)
    col_iota = lax.broadcasted_iota(jnp.int32, (_ROWS, _LANES), 1)
    valid = (row_iota * _LANES + col_iota) < a_valid

    scores = jax.nn.sigmoid(cls_ref[0])
    sc_ref[0] = jnp.where(valid, scores, _NEG_INF)

    aymin = anch_ref[0]
    axmin = anch_ref[1]
    aymax = anch_ref[2]
    axmax = anch_ref[3]
    ah = aymax - aymin
    aw = axmax - axmin
    acy = aymin + 0.5 * ah
    acx = axmin + 0.5 * aw
    dy = reg_ref[0, 0]
    dx = reg_ref[0, 1]
    dh = reg_ref[0, 2]
    dw = reg_ref[0, 3]
    pcy = dy * ah + acy
    pcx = dx * aw + acx
    ph = jnp.exp(dh) * ah
    pw = jnp.exp(dw) * aw
    y0_ref[0] = jnp.clip(pcy - 0.5 * ph, 0.0, 1.0)
    x0_ref[0] = jnp.clip(pcx - 0.5 * pw, 0.0, 1.0)
    y1_ref[0] = jnp.clip(pcy + 0.5 * ph, 0.0, 1.0)
    x1_ref[0] = jnp.clip(pcx + 0.5 * pw, 0.0, 1.0)


def _decode_tc(rpn_cls, rpn_reg, anchors, b, a_valid, interpret=False):
    pad = _A_PAD - a_valid
    cls_p = jnp.pad(rpn_cls[..., 0], ((0, 0), (0, pad))).reshape(
        b, _ROWS, _LANES)
    reg_p = jnp.pad(jnp.transpose(rpn_reg, (0, 2, 1)),
                    ((0, 0), (0, 0), (0, pad))).reshape(b, 4, _ROWS, _LANES)
    anch_p = jnp.pad(jnp.transpose(anchors, (1, 0)),
                     ((0, 0), (0, pad))).reshape(4, _ROWS, _LANES)

    plane = jax.ShapeDtypeStruct((b, _ROWS, _LANES), jnp.float32)
    outs = pl.pallas_call(
        functools.partial(_decode_body, a_valid),
        grid=(b,),
        in_specs=[
            pl.BlockSpec((1, _ROWS, _LANES), lambda i: (i, 0, 0)),
            pl.BlockSpec((1, 4, _ROWS, _LANES), lambda i: (i, 0, 0, 0)),
            pl.BlockSpec((4, _ROWS, _LANES), lambda i: (0, 0, 0)),
        ],
        out_specs=[pl.BlockSpec((1, _ROWS, _LANES), lambda i: (i, 0, 0))] * 5,
        out_shape=[plane] * 5,
        interpret=interpret,
    )(cls_p, reg_p, anch_p)
    return [o.reshape(b, _A_PAD) for o in outs]


def _axis_ids():
    return lax.axis_index("c"), lax.axis_index("s")


def _store1(ref, pos, val):
    plsc.store_scatter(ref, [jnp.full((16,), pos, jnp.int32)],
                       jnp.full((16,), val, jnp.float32),
                       mask=lax.iota(jnp.int32, 16) == 0)


def _sc_nms_body(top_n, iou_thr, n_samples,
                 sc_hbm, y0_hbm, x0_hbm, y1_hbm, x1_hbm, out_hbm,
                 sc_v, y0_v, x0_v, y1_v, x1_v, l1_v, l2_v,
                 ky0_v, kx0_v, ky1_v, kx1_v, kar_v, sem_a, sem_b):
    c, s = _axis_ids()
    n_cores = 2
    per_core = n_samples // n_cores

    @pl.when(s < per_core)
    def _work():
        samp = c * per_core + s
        h_sc = pltpu.async_copy(sc_hbm.at[samp], sc_v, sem_a)
        h_y0 = pltpu.async_copy(y0_hbm.at[samp], y0_v, sem_b)
        h_x0 = pltpu.async_copy(x0_hbm.at[samp], x0_v, sem_b)
        h_y1 = pltpu.async_copy(y1_hbm.at[samp], y1_v, sem_b)
        h_x1 = pltpu.async_copy(x1_hbm.at[samp], x1_v, sem_b)

        iota = lax.iota(jnp.int32, 16)
        zeros16 = jnp.zeros((16,), jnp.float32)

        for j in range(_KCAP // 16):
            ky0_v[pl.ds(j * 16, 16)] = zeros16
            kx0_v[pl.ds(j * 16, 16)] = zeros16
            ky1_v[pl.ds(j * 16, 16)] = zeros16
            kx1_v[pl.ds(j * 16, 16)] = zeros16
            kar_v[pl.ds(j * 16, 16)] = zeros16

        h_sc.wait()

        def l1_build(i, _):
            for u in range(4):
                ch = sc_v[pl.ds((i * 4 + u) * 16, 16)]
                _store1(l1_v, i * 4 + u, jnp.max(ch))
            return 0
        lax.fori_loop(0, _L1N // 4, l1_build, 0)

        def l2_build(i, _):
            for u in range(4):
                ch = l1_v[pl.ds((i * 4 + u) * 16, 16)]
                _store1(l2_v, i * 4 + u, jnp.max(ch))
            return 0
        lax.fori_loop(0, _L2N // 4, l2_build, 0)

        h_y0.wait()
        h_x0.wait()
        h_y1.wait()
        h_x1.wait()

        def wcond(state):
            kn, alive = state
            return (kn < top_n) & (alive > 0)

        def wbody(state):
            kn, alive = state

            l2chunks = [l2_v[pl.ds(k * 16, 16)] for k in range(_L2N // 16)]
            vmax = l2chunks[0]
            for ch in l2chunks[1:]:
                vmax = jnp.maximum(vmax, ch)
            m = jnp.max(vmax)

            cand = jnp.where(l2chunks[0] == m, iota, _BIG)
            for k, ch in enumerate(l2chunks[1:]):
                cand = jnp.minimum(
                    cand, jnp.where(ch == m, (k + 1) * 16 + iota, _BIG))
            p2 = jnp.min(cand)

            ch1 = l1_v[pl.ds(p2 * 16, 16)]
            f1 = plsc.all_reduce_ffs(ch1 == m)
            p1 = p2 * 16 + f1[0]

            ch0 = sc_v[pl.ds(p1 * 16, 16)]
            f0 = plsc.all_reduce_ffs(ch0 == m)
            hit0 = iota == f0

            live = m > _NEG_INF

            ch0n = jnp.where(hit0, _NEG_INF, ch0)
            sc_v[pl.ds(p1 * 16, 16)] = ch0n
            m1 = jnp.max(ch0n)
            _store1(l1_v, p1, m1)
            ch1n = jnp.where(iota == f1, m1, ch1)
            _store1(l2_v, p2, jnp.max(ch1n))

            pvec = p1 * 16 + f0
            cy0 = plsc.load_gather(y0_v, [pvec])
            cx0 = plsc.load_gather(x0_v, [pvec])
            cy1 = plsc.load_gather(y1_v, [pvec])
            cx1 = plsc.load_gather(x1_v, [pvec])
            car = (cy1 - cy0) * (cx1 - cx0)

            nkc = (kn + 15) // 16

            def ibody(j, sup):
                a0 = ky0_v[pl.ds(j * 16, 16)]
                b0 = kx0_v[pl.ds(j * 16, 16)]
                a1 = ky1_v[pl.ds(j * 16, 16)]
                b1 = kx1_v[pl.ds(j * 16, 16)]
                ar = kar_v[pl.ds(j * 16, 16)]
                yi0 = jnp.maximum(cy0, a0)
                xi0 = jnp.maximum(cx0, b0)
                yi1 = jnp.minimum(cy1, a1)
                xi1 = jnp.minimum(cx1, b1)
                inter = (jnp.maximum(yi1 - yi0, 0.0)
                         * jnp.maximum(xi1 - xi0, 0.0))
                iou = inter / (car + ar - inter + 1e-8)
                return sup | (iou > iou_thr).astype(jnp.int32)

            sup = lax.fori_loop(0, nkc, ibody, jnp.zeros((16,), jnp.int32))
            nsup = plsc.all_reduce_population_count(sup > 0)
            keep = live & (nsup[0] == 0)

            @pl.when(keep)
            def _append():
                _store1(ky0_v, kn, cy0[0])
                _store1(kx0_v, kn, cx0[0])
                _store1(ky1_v, kn, cy1[0])
                _store1(kx1_v, kn, cx1[0])
                _store1(kar_v, kn, car[0])

            kn = kn + jnp.where(keep, jnp.int32(1), jnp.int32(0))
            return (kn, jnp.where(live, jnp.int32(1), jnp.int32(0)))

        kn, _ = lax.while_loop(wcond, wbody,
                               (jnp.int32(0), jnp.int32(1)))

        b0y0 = jnp.full((16,), y0_v[pl.ds(0, 16)][0], jnp.float32)
        b0x0 = jnp.full((16,), x0_v[pl.ds(0, 16)][0], jnp.float32)
        b0y1 = jnp.full((16,), y1_v[pl.ds(0, 16)][0], jnp.float32)
        b0x1 = jnp.full((16,), x1_v[pl.ds(0, 16)][0], jnp.float32)
        for j in range(_KCAP // 16):
            kidx = j * 16 + iota
            mask = kidx >= kn
            ky0_v[pl.ds(j * 16, 16)] = jnp.where(
                mask, b0y0, ky0_v[pl.ds(j * 16, 16)])
            kx0_v[pl.ds(j * 16, 16)] = jnp.where(
                mask, b0x0, kx0_v[pl.ds(j * 16, 16)])
            ky1_v[pl.ds(j * 16, 16)] = jnp.where(
                mask, b0y1, ky1_v[pl.ds(j * 16, 16)])
            kx1_v[pl.ds(j * 16, 16)] = jnp.where(
                mask, b0x1, kx1_v[pl.ds(j * 16, 16)])

        pltpu.sync_copy(ky0_v, out_hbm.at[samp, 0])
        pltpu.sync_copy(kx0_v, out_hbm.at[samp, 1])
        pltpu.sync_copy(ky1_v, out_hbm.at[samp, 2])
        pltpu.sync_copy(kx1_v, out_hbm.at[samp, 3])


def _sc_nms(scores, y0, x0, y1, x1, b, top_n, iou_thr, interpret=False):
    mesh = plsc.VectorSubcoreMesh(core_axis_name="c", subcore_axis_name="s",
                                  num_cores=2, num_subcores=16)
    fn = pl.kernel(
        functools.partial(_sc_nms_body, top_n, iou_thr, b),
        out_type=jax.ShapeDtypeStruct((b, 4, _KCAP), jnp.float32),
        mesh=mesh,
        scratch_types=[
            pltpu.VMEM((_A_PAD,), jnp.float32),
            pltpu.VMEM((_A_PAD,), jnp.float32),
            pltpu.VMEM((_A_PAD,), jnp.float32),
            pltpu.VMEM((_A_PAD,), jnp.float32),
            pltpu.VMEM((_A_PAD,), jnp.float32),
            pltpu.VMEM((_L1N,), jnp.float32),
            pltpu.VMEM((_L2N,), jnp.float32),
            pltpu.VMEM((_KCAP,), jnp.float32),
            pltpu.VMEM((_KCAP,), jnp.float32),
            pltpu.VMEM((_KCAP,), jnp.float32),
            pltpu.VMEM((_KCAP,), jnp.float32),
            pltpu.VMEM((_KCAP,), jnp.float32),
            pltpu.SemaphoreType.DMA,
            pltpu.SemaphoreType.DMA,
        ],
        compiler_params=pltpu.CompilerParams(needs_layout_passes=False,
                                             skip_device_barrier=True),
        interpret=interpret,
    )
    return fn(scores, y0, x0, y1, x1)


def _proposal(rpn_cls, rpn_reg, anchors, top_n, iou_thr, interpret=False):
    b = rpn_cls.shape[0]
    a = rpn_cls.shape[1]
    scores, y0, x0, y1, x1 = _decode_tc(rpn_cls, rpn_reg, anchors, b, a,
                                        interpret=interpret)
    out = _sc_nms(scores, y0, x0, y1, x1, b, top_n, iou_thr,
                  interpret=interpret)
    proposals = jnp.transpose(out, (0, 2, 1))[:, :top_n, :].reshape(
        b * top_n, 4)
    indices = jnp.zeros((b * top_n,), jnp.int32)
    return proposals, indices


def kernel(rpn_cls, rpn_reg, anchors):
    return _proposal(rpn_cls, rpn_reg, anchors, _TOP_N, _IOU_THR)


# speedup vs baseline: 1.7133x; 1.0007x over previous
import functools

import jax
import jax.numpy as jnp
from jax import lax
from jax.experimental import pallas as pl
from jax.experimental.pallas import tpu as pltpu
from jax.experimental.pallas import tpu_sc as plsc

_A = 20000
_TOP_N = 200
_IOU_THR = 0.7
_LANES = 128

_A_PAD = 20480
_ROWS = _A_PAD // _LANES
_KCAP = 208
_L1N = _A_PAD // 16
_L2N = _L1N // 16
_BIG = 1 << 30
_NEG_INF = float("-inf")


def _decode_body(a_valid, cls_ref, reg_ref, anch_ref,
                 sc_ref, y0_ref, x0_ref, y1_ref, x1_ref):
    row_iota = lax.broadcasted_iota(jnp.int32, (_ROWS, _LANES), 0)
    col_iota = lax.broadcasted_iota(jnp.int32, (_ROWS, _LANES), 1)
    valid = (row_iota * _LANES + col_iota) < a_valid

    scores = jax.nn.sigmoid(cls_ref[0])
    sc_ref[0] = jnp.where(valid, scores, _NEG_INF)

    aymin = anch_ref[0]
    axmin = anch_ref[1]
    aymax = anch_ref[2]
    axmax = anch_ref[3]
    ah = aymax - aymin
    aw = axmax - axmin
    acy = aymin + 0.5 * ah
    acx = axmin + 0.5 * aw
    dy = reg_ref[0, 0]
    dx = reg_ref[0, 1]
    dh = reg_ref[0, 2]
    dw = reg_ref[0, 3]
    pcy = dy * ah + acy
    pcx = dx * aw + acx
    ph = jnp.exp(dh) * ah
    pw = jnp.exp(dw) * aw
    y0_ref[0] = jnp.clip(pcy - 0.5 * ph, 0.0, 1.0)
    x0_ref[0] = jnp.clip(pcx - 0.5 * pw, 0.0, 1.0)
    y1_ref[0] = jnp.clip(pcy + 0.5 * ph, 0.0, 1.0)
    x1_ref[0] = jnp.clip(pcx + 0.5 * pw, 0.0, 1.0)


def _decode_tc(rpn_cls, rpn_reg, anchors, b, a_valid, interpret=False):
    pad = _A_PAD - a_valid
    cls_p = jnp.pad(rpn_cls[..., 0], ((0, 0), (0, pad))).reshape(
        b, _ROWS, _LANES)
    reg_p = jnp.pad(jnp.transpose(rpn_reg, (0, 2, 1)),
                    ((0, 0), (0, 0), (0, pad))).reshape(b, 4, _ROWS, _LANES)
    anch_p = jnp.pad(jnp.transpose(anchors, (1, 0)),
                     ((0, 0), (0, pad))).reshape(4, _ROWS, _LANES)

    plane = jax.ShapeDtypeStruct((b, _ROWS, _LANES), jnp.float32)
    outs = pl.pallas_call(
        functools.partial(_decode_body, a_valid),
        grid=(b,),
        in_specs=[
            pl.BlockSpec((1, _ROWS, _LANES), lambda i: (i, 0, 0)),
            pl.BlockSpec((1, 4, _ROWS, _LANES), lambda i: (i, 0, 0, 0)),
            pl.BlockSpec((4, _ROWS, _LANES), lambda i: (0, 0, 0)),
        ],
        out_specs=[pl.BlockSpec((1, _ROWS, _LANES), lambda i: (i, 0, 0))] * 5,
        out_shape=[plane] * 5,
        interpret=interpret,
    )(cls_p, reg_p, anch_p)
    return [o.reshape(b, _A_PAD) for o in outs]


def _axis_ids():
    return lax.axis_index("c"), lax.axis_index("s")


def _store1(ref, pos, val):
    plsc.store_scatter(ref, [jnp.full((16,), pos, jnp.int32)],
                       jnp.full((16,), val, jnp.float32),
                       mask=lax.iota(jnp.int32, 16) == 0)


def _sc_nms_body(top_n, iou_thr, n_samples,
                 sc_hbm, y0_hbm, x0_hbm, y1_hbm, x1_hbm, out_hbm,
                 sc_v, y0_v, x0_v, y1_v, x1_v, l1_v, l2_v,
                 ky0_v, kx0_v, ky1_v, kx1_v, kar_v, sem_a, sem_b):
    c, s = _axis_ids()
    n_cores = 2
    per_core = n_samples // n_cores

    @pl.when(s < per_core)
    def _work():
        samp = c * per_core + s
        h_sc = pltpu.async_copy(sc_hbm.at[samp], sc_v, sem_a)
        h_y0 = pltpu.async_copy(y0_hbm.at[samp], y0_v, sem_b)
        h_x0 = pltpu.async_copy(x0_hbm.at[samp], x0_v, sem_b)
        h_y1 = pltpu.async_copy(y1_hbm.at[samp], y1_v, sem_b)
        h_x1 = pltpu.async_copy(x1_hbm.at[samp], x1_v, sem_b)

        iota = lax.iota(jnp.int32, 16)
        zeros16 = jnp.zeros((16,), jnp.float32)

        for j in range(_KCAP // 16):
            ky0_v[pl.ds(j * 16, 16)] = zeros16
            kx0_v[pl.ds(j * 16, 16)] = zeros16
            ky1_v[pl.ds(j * 16, 16)] = zeros16
            kx1_v[pl.ds(j * 16, 16)] = zeros16
            kar_v[pl.ds(j * 16, 16)] = zeros16

        h_sc.wait()

        def l1_build(i, _):
            for u in range(4):
                ch = sc_v[pl.ds((i * 4 + u) * 16, 16)]
                _store1(l1_v, i * 4 + u, jnp.max(ch))
            return 0
        lax.fori_loop(0, _L1N // 4, l1_build, 0)

        def l2_build(i, _):
            for u in range(4):
                ch = l1_v[pl.ds((i * 4 + u) * 16, 16)]
                _store1(l2_v, i * 4 + u, jnp.max(ch))
            return 0
        lax.fori_loop(0, _L2N // 4, l2_build, 0)

        h_y0.wait()
        h_x0.wait()
        h_y1.wait()
        h_x1.wait()

        def wcond(state):
            kn, alive = state
            return (kn < top_n) & (alive > 0)

        def wbody(state):
            kn, alive = state

            l2chunks = [l2_v[pl.ds(k * 16, 16)] for k in range(_L2N // 16)]
            vmax = l2chunks[0]
            for ch in l2chunks[1:]:
                vmax = jnp.maximum(vmax, ch)
            m = jnp.max(vmax)

            cand = jnp.where(l2chunks[0] == m, iota, _BIG)
            for k, ch in enumerate(l2chunks[1:]):
                cand = jnp.minimum(
                    cand, jnp.where(ch == m, (k + 1) * 16 + iota, _BIG))
            p2 = jnp.min(cand)

            ch1 = l1_v[pl.ds(p2 * 16, 16)]
            f1 = plsc.all_reduce_ffs(ch1 == m)
            p1 = p2 * 16 + f1[0]

            ch0 = sc_v[pl.ds(p1 * 16, 16)]
            f0 = plsc.all_reduce_ffs(ch0 == m)
            hit0 = iota == f0

            live = m > _NEG_INF

            ch0n = jnp.where(hit0, _NEG_INF, ch0)
            sc_v[pl.ds(p1 * 16, 16)] = ch0n
            m1 = jnp.max(ch0n)
            _store1(l1_v, p1, m1)
            ch1n = jnp.where(iota == f1, m1, ch1)
            _store1(l2_v, p2, jnp.max(ch1n))

            pvec = p1 * 16 + f0
            cy0 = plsc.load_gather(y0_v, [pvec])
            cx0 = plsc.load_gather(x0_v, [pvec])
            cy1 = plsc.load_gather(y1_v, [pvec])
            cx1 = plsc.load_gather(x1_v, [pvec])
            car = (cy1 - cy0) * (cx1 - cx0)

            nkc = (kn + 15) // 16

            def ibody(j, sup):
                a0 = ky0_v[pl.ds(j * 16, 16)]
                b0 = kx0_v[pl.ds(j * 16, 16)]
                a1 = ky1_v[pl.ds(j * 16, 16)]
                b1 = kx1_v[pl.ds(j * 16, 16)]
                ar = kar_v[pl.ds(j * 16, 16)]
                yi0 = jnp.maximum(cy0, a0)
                xi0 = jnp.maximum(cx0, b0)
                yi1 = jnp.minimum(cy1, a1)
                xi1 = jnp.minimum(cx1, b1)
                inter = (jnp.maximum(yi1 - yi0, 0.0)
                         * jnp.maximum(xi1 - xi0, 0.0))
                iou = inter / (car + ar - inter + 1e-8)
                return sup | (iou > iou_thr).astype(jnp.int32)

            sup = lax.fori_loop(0, nkc, ibody, jnp.zeros((16,), jnp.int32))
            nsup = plsc.all_reduce_population_count(sup > 0)
            keep = live & (nsup[0] == 0)

            @pl.when(keep)
            def _append():
                _store1(ky0_v, kn, cy0[0])
                _store1(kx0_v, kn, cx0[0])
                _store1(ky1_v, kn, cy1[0])
                _store1(kx1_v, kn, cx1[0])
                _store1(kar_v, kn, car[0])

            kn = kn + jnp.where(keep, jnp.int32(1), jnp.int32(0))
            return (kn, jnp.where(live, jnp.int32(1), jnp.int32(0)))

        kn, _ = lax.while_loop(wcond, wbody,
                               (jnp.int32(0), jnp.int32(1)))

        b0y0 = jnp.full((16,), y0_v[pl.ds(0, 16)][0], jnp.float32)
        b0x0 = jnp.full((16,), x0_v[pl.ds(0, 16)][0], jnp.float32)
        b0y1 = jnp.full((16,), y1_v[pl.ds(0, 16)][0], jnp.float32)
        b0x1 = jnp.full((16,), x1_v[pl.ds(0, 16)][0], jnp.float32)
        for j in range(_KCAP // 16):
            kidx = j * 16 + iota
            mask = kidx >= kn
            ky0_v[pl.ds(j * 16, 16)] = jnp.where(
                mask, b0y0, ky0_v[pl.ds(j * 16, 16)])
            kx0_v[pl.ds(j * 16, 16)] = jnp.where(
                mask, b0x0, kx0_v[pl.ds(j * 16, 16)])
            ky1_v[pl.ds(j * 16, 16)] = jnp.where(
                mask, b0y1, ky1_v[pl.ds(j * 16, 16)])
            kx1_v[pl.ds(j * 16, 16)] = jnp.where(
                mask, b0x1, kx1_v[pl.ds(j * 16, 16)])

        pltpu.sync_copy(ky0_v, out_hbm.at[samp, 0])
        pltpu.sync_copy(kx0_v, out_hbm.at[samp, 1])
        pltpu.sync_copy(ky1_v, out_hbm.at[samp, 2])
        pltpu.sync_copy(kx1_v, out_hbm.at[samp, 3])


def _sc_nms(scores, y0, x0, y1, x1, b, top_n, iou_thr, interpret=False):
    mesh = plsc.VectorSubcoreMesh(core_axis_name="c", subcore_axis_name="s",
                                  num_cores=2, num_subcores=16)
    fn = pl.kernel(
        functools.partial(_sc_nms_body, top_n, iou_thr, b),
        out_type=jax.ShapeDtypeStruct((b, 4, _KCAP), jnp.float32),
        mesh=mesh,
        scratch_types=[
            pltpu.VMEM((_A_PAD,), jnp.float32),
            pltpu.VMEM((_A_PAD,), jnp.float32),
            pltpu.VMEM((_A_PAD,), jnp.float32),
            pltpu.VMEM((_A_PAD,), jnp.float32),
            pltpu.VMEM((_A_PAD,), jnp.float32),
            pltpu.VMEM((_L1N,), jnp.float32),
            pltpu.VMEM((_L2N,), jnp.float32),
            pltpu.VMEM((_KCAP,), jnp.float32),
            pltpu.VMEM((_KCAP,), jnp.float32),
            pltpu.VMEM((_KCAP,), jnp.float32),
            pltpu.VMEM((_KCAP,), jnp.float32),
            pltpu.VMEM((_KCAP,), jnp.float32),
            pltpu.SemaphoreType.DMA,
            pltpu.SemaphoreType.DMA,
        ],
        compiler_params=pltpu.CompilerParams(needs_layout_passes=False),
        interpret=interpret,
    )
    return fn(scores, y0, x0, y1, x1)


def _proposal(rpn_cls, rpn_reg, anchors, top_n, iou_thr, interpret=False):
    b = rpn_cls.shape[0]
    a = rpn_cls.shape[1]
    scores, y0, x0, y1, x1 = _decode_tc(rpn_cls, rpn_reg, anchors, b, a,
                                        interpret=interpret)
    out = _sc_nms(scores, y0, x0, y1, x1, b, top_n, iou_thr,
                  interpret=interpret)
    proposals = jnp.transpose(out, (0, 2, 1))[:, :top_n, :].reshape(
        b * top_n, 4)
    indices = jnp.zeros((b * top_n,), jnp.int32)
    return proposals, indices


def kernel(rpn_cls, rpn_reg, anchors):
    return _proposal(rpn_cls, rpn_reg, anchors, _TOP_N, _IOU_THR)
